# baseline (speedup 1.0000x reference)
import functools

import jax
import jax.numpy as jnp
from jax import lax
from jax.experimental import pallas as pl
from jax.experimental.pallas import tpu as pltpu
from jax.experimental.pallas import tpu_sc as plsc

_GATHER = 128


def _sc_gather_concat(idxs, tables, T, D, R):
    info = plsc.get_sparse_core_info()
    NW = info.num_cores * info.num_subcores
    rows_per_w = R // NW
    n_chunks = rows_per_w // _GATHER

    mesh = plsc.VectorSubcoreMesh(core_axis_name="c", subcore_axis_name="s")

    @functools.partial(
        pl.kernel,
        mesh=mesh,
        out_type=jax.ShapeDtypeStruct((R, T, D), jnp.float32),
        scratch_types=[
            pltpu.VMEM((2, T, _GATHER), jnp.int32),
            pltpu.VMEM((2, T, _GATHER, D), jnp.float32),
            pltpu.SemaphoreType.DMA,
            pltpu.SemaphoreType.DMA,
            pltpu.SemaphoreType.DMA,
            pltpu.SemaphoreType.DMA,
        ],
    )
    def k(i0, i1, i2, w0, w1, w2, out_hbm, idxvm, rows, g0, g1, s0, s1):
        idx_hbm = (i0, i1, i2)
        w_hbm = (w0, w1, w2)
        wid = lax.axis_index("s") * info.num_cores + lax.axis_index("c")
        rbase0 = wid * rows_per_w
        gsem = (g0, g1)
        ssem = (s0, s1)

        def load_idx(c, b):
            rbase = rbase0 + c * _GATHER
            for t in range(T):
                pltpu.sync_copy(
                    idx_hbm[t].at[pl.ds(rbase, _GATHER)], idxvm.at[b].at[t]
                )

        def fire_gathers(b):
            for t in range(T):
                pltpu.async_copy(
                    w_hbm[t].at[idxvm.at[b].at[t]], rows.at[b].at[t], gsem[b]
                )

        def wait_gathers(b):
            for t in range(T):
                pltpu.make_async_copy(
                    w_hbm[t].at[idxvm.at[b].at[t]], rows.at[b].at[t], gsem[b]
                ).wait()

        def chunk_op(c, b, prefetch):
            wait_gathers(b)
            rbase = rbase0 + c * _GATHER
            scs = [
                pltpu.async_copy(
                    rows.at[b].at[t].reshape(_GATHER, 1, D),
                    out_hbm.at[pl.ds(rbase, _GATHER), pl.ds(t, 1), :],
                    ssem[b],
                )
                for t in range(T)
            ]
            if prefetch:
                load_idx(c + 2, b)
            for sc in scs:
                sc.wait()
            if prefetch:
                fire_gathers(b)

        for b in range(2):
            load_idx(b, b)
            fire_gathers(b)

        def body(kk, carry):
            for b in range(2):
                chunk_op(2 * kk + b, b, True)
            return carry

        lax.fori_loop(0, n_chunks // 2 - 1, body, 0)
        for b in range(2):
            chunk_op(n_chunks - 2 + b, b, False)

    return k(*idxs, *tables)


def kernel(bbs_inf, phi_W, psi_W, omega_W):
    B, L, T = bbs_inf.shape
    V, D = phi_W.shape
    R = B * L

    idxs = [bbs_inf[:, :, t].reshape(R) for t in range(T)]
    out = _sc_gather_concat(idxs, (phi_W, psi_W, omega_W), T, D, R)
    return out.reshape(B, L, T * D)


# speedup vs baseline: 2.5461x; 2.5461x over previous
import functools

import jax
import jax.numpy as jnp
from jax import lax
from jax.experimental import pallas as pl
from jax.experimental.pallas import tpu as pltpu
from jax.experimental.pallas import tpu_sc as plsc

_GATHER = 128


def _sc_gather_concat(idxs, tables, T, D, R):
    info = plsc.get_sparse_core_info()
    NW = info.num_cores * info.num_subcores
    rows_per_w = R // NW
    n_chunks = rows_per_w // _GATHER

    mesh = plsc.VectorSubcoreMesh(core_axis_name="c", subcore_axis_name="s")

    @functools.partial(
        pl.kernel,
        mesh=mesh,
        out_type=jax.ShapeDtypeStruct((R, T * D), jnp.float32),
        scratch_types=[
            pltpu.VMEM((2, T, _GATHER), jnp.int32),
            pltpu.VMEM((2, T, _GATHER, D), jnp.float32),
            pltpu.SemaphoreType.DMA,
            pltpu.SemaphoreType.DMA,
            pltpu.SemaphoreType.DMA,
            pltpu.SemaphoreType.DMA,
        ],
    )
    def k(i0, i1, i2, w0, w1, w2, out_hbm, idxvm, rows, g0, g1, s0, s1):
        idx_hbm = (i0, i1, i2)
        w_hbm = (w0, w1, w2)
        wid = lax.axis_index("s") * info.num_cores + lax.axis_index("c")
        rbase0 = wid * rows_per_w
        gsem = (g0, g1)
        ssem = (s0, s1)

        def load_idx(c, b):
            rbase = rbase0 + c * _GATHER
            for t in range(T):
                pltpu.sync_copy(
                    idx_hbm[t].at[pl.ds(rbase, _GATHER)], idxvm.at[b].at[t]
                )

        def fire_gathers(b):
            for t in range(T):
                pltpu.async_copy(
                    w_hbm[t].at[idxvm.at[b].at[t]], rows.at[b].at[t], gsem[b]
                )

        def wait_gathers(b):
            for t in range(T):
                pltpu.make_async_copy(
                    w_hbm[t].at[idxvm.at[b].at[t]], rows.at[b].at[t], gsem[b]
                ).wait()

        def chunk_op(c, b, prefetch):
            wait_gathers(b)
            rbase = rbase0 + c * _GATHER
            scs = [
                pltpu.async_copy(
                    rows.at[b].at[t],
                    out_hbm.at[pl.ds(rbase, _GATHER), pl.ds(t * D, D)],
                    ssem[b],
                )
                for t in range(T)
            ]
            if prefetch:
                load_idx(c + 2, b)
            for sc in scs:
                sc.wait()
            if prefetch:
                fire_gathers(b)

        for b in range(2):
            load_idx(b, b)
            fire_gathers(b)

        def body(kk, carry):
            for b in range(2):
                chunk_op(2 * kk + b, b, True)
            return carry

        lax.fori_loop(0, n_chunks // 2 - 1, body, 0)
        for b in range(2):
            chunk_op(n_chunks - 2 + b, b, False)

    return k(*idxs, *tables)


def kernel(bbs_inf, phi_W, psi_W, omega_W):
    B, L, T = bbs_inf.shape
    V, D = phi_W.shape
    R = B * L

    idxs = [bbs_inf[:, :, t].reshape(R) for t in range(T)]
    out = _sc_gather_concat(idxs, (phi_W, psi_W, omega_W), T, D, R)
    return out.reshape(B, L, T * D)
